# Initial kernel scaffold; baseline (speedup 1.0000x reference)
import functools

import jax
import jax.numpy as jnp
from jax import lax
from jax.experimental import pallas as pl
from jax.experimental.pallas import tpu as pltpu
from jax.experimental.pallas import tpu_sc as plsc

NUM_CORES = 2
NUM_SUBCORES = 16
NUM_WORKERS = NUM_CORES * NUM_SUBCORES
LANES = 16
CHUNK = 128


@functools.partial(jax.jit, static_argnums=(3, 4))
def _gather_call(table, idx, mask, n, d):
    per_worker = n // NUM_WORKERS
    n_chunks = per_worker // CHUNK
    mesh = plsc.VectorSubcoreMesh(core_axis_name="c", subcore_axis_name="s")

    @functools.partial(
        pl.kernel,
        out_type=jax.ShapeDtypeStruct((n, d), jnp.float32),
        mesh=mesh,
        scratch_types=[
            pltpu.VMEM((CHUNK,), jnp.int32),
            pltpu.VMEM((CHUNK,), jnp.float32),
            pltpu.VMEM((CHUNK, d), jnp.float32),
            pltpu.SemaphoreType.DMA,
        ],
    )
    def body(table_hbm, idx_hbm, mask_hbm, out_hbm, idx_v, mask_v, rows_v, sem):
        wid = lax.axis_index("c") * NUM_SUBCORES + lax.axis_index("s")
        base = wid * per_worker

        def chunk_body(i, carry):
            start = base + i * CHUNK
            pltpu.sync_copy(idx_hbm.at[pl.ds(start, CHUNK)], idx_v)
            pltpu.sync_copy(mask_hbm.at[pl.ds(start, CHUNK)], mask_v)
            pltpu.async_copy(table_hbm.at[idx_v], rows_v, sem).wait()

            for g in range(CHUNK // LANES):
                iv = idx_v[pl.ds(g * LANES, LANES)]
                mv = mask_v[pl.ds(g * LANES, LANES)]
                scale = jnp.where(iv == 0, 0.0, mv)
                bad = scale != 1.0

                @pl.when(jnp.any(bad))
                def _fix(g=g, scale=scale, bad=bad):
                    row_ids = g * LANES + lax.iota(jnp.int32, LANES)

                    def fix_col(k, c):
                        col = jnp.full((LANES,), k, jnp.int32)
                        v = plsc.load_gather(rows_v, [row_ids, col])
                        plsc.store_scatter(
                            rows_v, [row_ids, col], v * scale, mask=bad
                        )
                        return c

                    lax.fori_loop(0, d, fix_col, 0)

            pltpu.sync_copy(rows_v, out_hbm.at[pl.ds(start, CHUNK)])
            return carry

        lax.fori_loop(0, n_chunks, chunk_body, 0)

    return body(table, idx, mask)


def kernel(input_ids, attention_mask, table):
    b, l = input_ids.shape
    d = table.shape[1]
    n = b * l
    idx = input_ids.reshape(n).astype(jnp.int32)
    mask = attention_mask.reshape(n).astype(jnp.float32)
    out = _gather_call(table, idx, mask, n, d)
    return out.reshape(b, l, d)


# speedup vs baseline: 4.1524x; 4.1524x over previous
import functools

import jax
import jax.numpy as jnp
from jax import lax
from jax.experimental import pallas as pl
from jax.experimental.pallas import tpu as pltpu
from jax.experimental.pallas import tpu_sc as plsc

NUM_CORES = 2
NUM_SUBCORES = 16
NUM_WORKERS = NUM_CORES * NUM_SUBCORES
LANES = 16
CHUNK = 128


@functools.partial(jax.jit, static_argnums=(3, 4))
def _gather_call(table, idx, mask, n, d):
    per_worker = n // NUM_WORKERS
    n_chunks = per_worker // CHUNK
    mesh = plsc.VectorSubcoreMesh(core_axis_name="c", subcore_axis_name="s")

    @functools.partial(
        pl.kernel,
        out_type=jax.ShapeDtypeStruct((n, d), jnp.float32),
        mesh=mesh,
        scratch_types=[
            pltpu.VMEM((CHUNK,), jnp.int32),
            pltpu.VMEM((CHUNK,), jnp.float32),
            pltpu.VMEM((CHUNK, d), jnp.float32),
            pltpu.SemaphoreType.DMA,
        ],
        compiler_params=pltpu.CompilerParams(needs_layout_passes=False),
    )
    def body(table_hbm, idx_hbm, mask_hbm, out_hbm, idx_v, mask_v, rows_v, sem):
        wid = lax.axis_index("c") * NUM_SUBCORES + lax.axis_index("s")
        base = wid * per_worker

        def chunk_body(i, carry):
            start = base + i * CHUNK
            pltpu.sync_copy(idx_hbm.at[pl.ds(start, CHUNK)], idx_v)
            pltpu.sync_copy(mask_hbm.at[pl.ds(start, CHUNK)], mask_v)
            pltpu.async_copy(table_hbm.at[idx_v], rows_v, sem).wait()

            for g in range(CHUNK // LANES):
                iv = idx_v[pl.ds(g * LANES, LANES)]
                mv = mask_v[pl.ds(g * LANES, LANES)]
                scale = jnp.where(iv == 0, 0.0, mv)
                bad = scale != 1.0
                nbad = plsc.all_reduce_population_count(bad)

                @pl.when(nbad[0] > 0)
                def _fix(g=g, scale=scale, bad=bad):
                    row_ids = g * LANES + lax.iota(jnp.int32, LANES)

                    def fix_col(k, c):
                        col = jnp.full((LANES,), k, jnp.int32)
                        v = plsc.load_gather(rows_v, [row_ids, col])
                        plsc.store_scatter(
                            rows_v, [row_ids, col], v * scale, mask=bad
                        )
                        return c

                    lax.fori_loop(0, d, fix_col, 0)

            pltpu.sync_copy(rows_v, out_hbm.at[pl.ds(start, CHUNK)])
            return carry

        lax.fori_loop(0, n_chunks, chunk_body, 0)

    return body(table, idx, mask)


def kernel(input_ids, attention_mask, table):
    b, l = input_ids.shape
    d = table.shape[1]
    n = b * l
    idx = input_ids.reshape(n).astype(jnp.int32)
    mask = attention_mask.reshape(n).astype(jnp.float32)
    out = _gather_call(table, idx, mask, n, d)
    return out.reshape(b, l, d)


# speedup vs baseline: 6.8004x; 1.6377x over previous
import functools

import jax
import jax.numpy as jnp
from jax import lax
from jax.experimental import pallas as pl
from jax.experimental.pallas import tpu as pltpu
from jax.experimental.pallas import tpu_sc as plsc

NUM_CORES = 2
NUM_SUBCORES = 16
NUM_WORKERS = NUM_CORES * NUM_SUBCORES
LANES = 16
CHUNK = 128


def _fixup(rows_v, idx_row, mask_row, d):
    for g in range(CHUNK // LANES):
        iv = idx_row[pl.ds(g * LANES, LANES)]
        mv = mask_row[pl.ds(g * LANES, LANES)]
        scale = jnp.where(iv == 0, 0.0, mv)
        bad = scale != 1.0
        nbad = plsc.all_reduce_population_count(bad)

        @pl.when(nbad[0] > 0)
        def _fix(g=g, scale=scale, bad=bad):
            row_ids = g * LANES + lax.iota(jnp.int32, LANES)

            def fix_col(k, c):
                col = jnp.full((LANES,), k, jnp.int32)
                v = plsc.load_gather(rows_v, [row_ids, col])
                plsc.store_scatter(rows_v, [row_ids, col], v * scale, mask=bad)
                return c

            lax.fori_loop(0, d, fix_col, 0)


@functools.partial(jax.jit, static_argnums=(3, 4))
def _gather_call(table, idx, mask, n, d):
    per_worker = n // NUM_WORKERS
    n_chunks = per_worker // CHUNK
    mesh = plsc.VectorSubcoreMesh(core_axis_name="c", subcore_axis_name="s")

    @functools.partial(
        pl.kernel,
        out_type=jax.ShapeDtypeStruct((n, d), jnp.float32),
        mesh=mesh,
        scratch_types=[
            pltpu.VMEM((n_chunks, 1, CHUNK), jnp.int32),
            pltpu.VMEM((n_chunks, 1, CHUNK), jnp.float32),
            pltpu.VMEM((CHUNK, d), jnp.float32),
            pltpu.VMEM((CHUNK, d), jnp.float32),
            pltpu.SemaphoreType.DMA,
            pltpu.SemaphoreType.DMA,
        ],
        compiler_params=pltpu.CompilerParams(needs_layout_passes=False),
    )
    def body(table_hbm, idx_hbm, mask_hbm, out_hbm,
             idx_v, mask_v, rows0, rows1, g0, g1):
        wid = lax.axis_index("c") * NUM_SUBCORES + lax.axis_index("s")
        base = wid * per_worker
        bufs = (rows0, rows1)
        sems = (g0, g1)

        pltpu.sync_copy(idx_hbm.at[pl.ds(wid * n_chunks, n_chunks)], idx_v)
        pltpu.sync_copy(mask_hbm.at[pl.ds(wid * n_chunks, n_chunks)], mask_v)

        pltpu.async_copy(table_hbm.at[idx_v.at[0, 0]], rows0, g0)

        @pl.loop(0, n_chunks, step=2)
        def _outer(i):
            for b in range(2):
                c = i + b

                pltpu.make_async_copy(
                    table_hbm.at[idx_v.at[c, 0]], bufs[b], sems[b]
                ).wait()

                @pl.when(c + 1 < n_chunks)
                def _start(c=c, b=b):
                    pltpu.async_copy(
                        table_hbm.at[idx_v.at[c + 1, 0]], bufs[1 - b], sems[1 - b]
                    )

                _fixup(bufs[b], idx_v.at[c, 0], mask_v.at[c, 0], d)

                pltpu.sync_copy(
                    bufs[b], out_hbm.at[pl.ds(base + c * CHUNK, CHUNK)]
                )

    return body(table, idx, mask)


def kernel(input_ids, attention_mask, table):
    b, l = input_ids.shape
    d = table.shape[1]
    n = b * l
    idx = input_ids.reshape(n // CHUNK, 1, CHUNK).astype(jnp.int32)
    mask = attention_mask.reshape(n // CHUNK, 1, CHUNK).astype(jnp.float32)
    out = _gather_call(table, idx, mask, n, d)
    return out.reshape(b, l, d)


# speedup vs baseline: 7.9047x; 1.1624x over previous
import functools

import jax
import jax.numpy as jnp
from jax import lax
from jax.experimental import pallas as pl
from jax.experimental.pallas import tpu as pltpu
from jax.experimental.pallas import tpu_sc as plsc

NUM_CORES = 2
NUM_SUBCORES = 16
NUM_WORKERS = NUM_CORES * NUM_SUBCORES
LANES = 16
CHUNK = 128


def _fixup(rows_v, idx_row, mask_row, d):
    for g in range(CHUNK // LANES):
        iv = idx_row[pl.ds(g * LANES, LANES)]
        mv = mask_row[pl.ds(g * LANES, LANES)]
        scale = jnp.where(iv == 0, 0.0, mv)
        bad = scale != 1.0
        nbad = plsc.all_reduce_population_count(bad)

        @pl.when(nbad[0] > 0)
        def _fix(g=g, scale=scale, bad=bad):
            row_ids = g * LANES + lax.iota(jnp.int32, LANES)

            def fix_col(k, c):
                col = jnp.full((LANES,), k, jnp.int32)
                v = plsc.load_gather(rows_v, [row_ids, col])
                plsc.store_scatter(rows_v, [row_ids, col], v * scale, mask=bad)
                return c

            lax.fori_loop(0, d, fix_col, 0)


@functools.partial(jax.jit, static_argnums=(3, 4))
def _gather_call(table, idx, mask, n, d):
    per_worker = n // NUM_WORKERS
    n_chunks = per_worker // CHUNK
    mesh = plsc.VectorSubcoreMesh(core_axis_name="c", subcore_axis_name="s")

    nbuf = 5
    assert n_chunks % nbuf == 0

    @functools.partial(
        pl.kernel,
        out_type=jax.ShapeDtypeStruct((n, d), jnp.float32),
        mesh=mesh,
        scratch_types=[
            pltpu.VMEM((n_chunks, 1, CHUNK), jnp.int32),
            pltpu.VMEM((n_chunks, 1, CHUNK), jnp.float32),
            [pltpu.VMEM((CHUNK, d), jnp.float32)] * nbuf,
            [pltpu.SemaphoreType.DMA] * nbuf,
            [pltpu.SemaphoreType.DMA] * nbuf,
        ],
        compiler_params=pltpu.CompilerParams(needs_layout_passes=False),
    )
    def body(table_hbm, idx_hbm, mask_hbm, out_hbm,
             idx_v, mask_v, bufs, gsems, osems):
        wid = lax.axis_index("c") * NUM_SUBCORES + lax.axis_index("s")
        base = wid * per_worker

        pltpu.sync_copy(idx_hbm.at[pl.ds(wid * n_chunks, n_chunks)], idx_v)
        pltpu.sync_copy(mask_hbm.at[pl.ds(wid * n_chunks, n_chunks)], mask_v)

        for b in range(nbuf - 1):
            pltpu.async_copy(table_hbm.at[idx_v.at[b, 0]], bufs[b], gsems[b])

        @pl.loop(0, n_chunks, step=nbuf)
        def _outer(i):
            for b in range(nbuf):
                c = i + b

                pltpu.make_async_copy(
                    table_hbm.at[idx_v.at[c, 0]], bufs[b], gsems[b]
                ).wait()

                _fixup(bufs[b], idx_v.at[c, 0], mask_v.at[c, 0], d)

                pltpu.async_copy(
                    bufs[b], out_hbm.at[pl.ds(base + c * CHUNK, CHUNK)],
                    osems[b],
                )

                b2 = (b + nbuf - 1) % nbuf

                @pl.when(c + nbuf - 1 < n_chunks)
                def _start(c=c, b2=b2):
                    @pl.when(c >= 1)
                    def _wait_wb():
                        pltpu.make_async_copy(
                            bufs[b2],
                            out_hbm.at[pl.ds(base + (c - 1) * CHUNK, CHUNK)],
                            osems[b2],
                        ).wait()

                    pltpu.async_copy(
                        table_hbm.at[idx_v.at[c + nbuf - 1, 0]],
                        bufs[b2], gsems[b2],
                    )

        for b in range(nbuf):
            c_last = n_chunks - nbuf + b
            pltpu.make_async_copy(
                bufs[b], out_hbm.at[pl.ds(base + c_last * CHUNK, CHUNK)],
                osems[b],
            ).wait()

    return body(table, idx, mask)


def kernel(input_ids, attention_mask, table):
    b, l = input_ids.shape
    d = table.shape[1]
    n = b * l
    idx = input_ids.reshape(n // CHUNK, 1, CHUNK).astype(jnp.int32)
    mask = attention_mask.reshape(n // CHUNK, 1, CHUNK).astype(jnp.float32)
    out = _gather_call(table, idx, mask, n, d)
    return out.reshape(b, l, d)


# speedup vs baseline: 7.9292x; 1.0031x over previous
import functools

import jax
import jax.numpy as jnp
from jax import lax
from jax.experimental import pallas as pl
from jax.experimental.pallas import tpu as pltpu
from jax.experimental.pallas import tpu_sc as plsc

NUM_CORES = 2
NUM_SUBCORES = 16
NUM_WORKERS = NUM_CORES * NUM_SUBCORES
LANES = 16
CHUNK = 128


def _scale_of(idx_row, mask_row, g):
    iv = idx_row[pl.ds(g * LANES, LANES)]
    mv = mask_row[pl.ds(g * LANES, LANES)]
    return jnp.where(iv == 0, 0.0, mv)


def _fixup(rows_v, idx_row, mask_row, d):
    anybad = None
    for g in range(CHUNK // LANES):
        bad = _scale_of(idx_row, mask_row, g) != 1.0
        anybad = bad if anybad is None else anybad | bad
    nbad = plsc.all_reduce_population_count(anybad)

    @pl.when(nbad[0] > 0)
    def _fix_chunk():
        for g in range(CHUNK // LANES):
            scale = _scale_of(idx_row, mask_row, g)
            bad = scale != 1.0
            ngroup = plsc.all_reduce_population_count(bad)

            @pl.when(ngroup[0] > 0)
            def _fix(g=g, scale=scale, bad=bad):
                row_ids = g * LANES + lax.iota(jnp.int32, LANES)

                def fix_col(k, c):
                    col = jnp.full((LANES,), k, jnp.int32)
                    v = plsc.load_gather(rows_v, [row_ids, col])
                    plsc.store_scatter(
                        rows_v, [row_ids, col], v * scale, mask=bad
                    )
                    return c

                lax.fori_loop(0, d, fix_col, 0)


@functools.partial(jax.jit, static_argnums=(3, 4))
def _gather_call(table, idx, mask, n, d):
    per_worker = n // NUM_WORKERS
    n_chunks = per_worker // CHUNK
    mesh = plsc.VectorSubcoreMesh(core_axis_name="c", subcore_axis_name="s")

    nbuf = 5
    assert n_chunks % nbuf == 0

    @functools.partial(
        pl.kernel,
        out_type=jax.ShapeDtypeStruct((n, d), jnp.float32),
        mesh=mesh,
        scratch_types=[
            pltpu.VMEM((n_chunks, 1, CHUNK), jnp.int32),
            pltpu.VMEM((n_chunks, 1, CHUNK), jnp.float32),
            [pltpu.VMEM((CHUNK, d), jnp.float32)] * nbuf,
            [pltpu.SemaphoreType.DMA] * nbuf,
            [pltpu.SemaphoreType.DMA] * nbuf,
        ],
        compiler_params=pltpu.CompilerParams(needs_layout_passes=False),
    )
    def body(table_hbm, idx_hbm, mask_hbm, out_hbm,
             idx_v, mask_v, bufs, gsems, osems):
        wid = lax.axis_index("c") * NUM_SUBCORES + lax.axis_index("s")
        base = wid * per_worker

        pltpu.sync_copy(idx_hbm.at[pl.ds(wid * n_chunks, n_chunks)], idx_v)
        pltpu.sync_copy(mask_hbm.at[pl.ds(wid * n_chunks, n_chunks)], mask_v)

        for b in range(nbuf - 1):
            pltpu.async_copy(table_hbm.at[idx_v.at[b, 0]], bufs[b], gsems[b])

        @pl.loop(0, n_chunks, step=nbuf)
        def _outer(i):
            for b in range(nbuf):
                c = i + b

                b2 = (b + nbuf - 1) % nbuf

                @pl.when(c + nbuf - 1 < n_chunks)
                def _start(c=c, b2=b2):
                    @pl.when(c >= 1)
                    def _wait_wb():
                        pltpu.make_async_copy(
                            bufs[b2],
                            out_hbm.at[pl.ds(base + (c - 1) * CHUNK, CHUNK)],
                            osems[b2],
                        ).wait()

                    pltpu.async_copy(
                        table_hbm.at[idx_v.at[c + nbuf - 1, 0]],
                        bufs[b2], gsems[b2],
                    )

                pltpu.make_async_copy(
                    table_hbm.at[idx_v.at[c, 0]], bufs[b], gsems[b]
                ).wait()

                _fixup(bufs[b], idx_v.at[c, 0], mask_v.at[c, 0], d)

                pltpu.async_copy(
                    bufs[b], out_hbm.at[pl.ds(base + c * CHUNK, CHUNK)],
                    osems[b],
                )

        for b in range(nbuf):
            c_last = n_chunks - nbuf + b
            pltpu.make_async_copy(
                bufs[b], out_hbm.at[pl.ds(base + c_last * CHUNK, CHUNK)],
                osems[b],
            ).wait()

    return body(table, idx, mask)


def kernel(input_ids, attention_mask, table):
    b, l = input_ids.shape
    d = table.shape[1]
    n = b * l
    idx = input_ids.reshape(n // CHUNK, 1, CHUNK).astype(jnp.int32)
    mask = attention_mask.reshape(n // CHUNK, 1, CHUNK).astype(jnp.float32)
    out = _gather_call(table, idx, mask, n, d)
    return out.reshape(b, l, d)


# speedup vs baseline: 7.9321x; 1.0004x over previous
import functools

import jax
import jax.numpy as jnp
from jax import lax
from jax.experimental import pallas as pl
from jax.experimental.pallas import tpu as pltpu
from jax.experimental.pallas import tpu_sc as plsc

NUM_CORES = 2
NUM_SUBCORES = 16
NUM_WORKERS = NUM_CORES * NUM_SUBCORES
LANES = 16
CHUNK = 128


def _scale_of(idx_row, mask_row, g):
    iv = idx_row[pl.ds(g * LANES, LANES)]
    mv = mask_row[pl.ds(g * LANES, LANES)]
    return jnp.where(iv == 0, 0.0, mv)


def _fixup(rows_v, idx_row, mask_row, d):
    anybad = None
    for g in range(CHUNK // LANES):
        bad = _scale_of(idx_row, mask_row, g) != 1.0
        anybad = bad if anybad is None else anybad | bad
    nbad = plsc.all_reduce_population_count(anybad)

    @pl.when(nbad[0] > 0)
    def _fix_chunk():
        for g in range(CHUNK // LANES):
            scale = _scale_of(idx_row, mask_row, g)
            bad = scale != 1.0
            ngroup = plsc.all_reduce_population_count(bad)

            @pl.when(ngroup[0] > 0)
            def _fix(g=g, scale=scale, bad=bad):
                row_ids = g * LANES + lax.iota(jnp.int32, LANES)

                def fix_col(k, c):
                    col = jnp.full((LANES,), k, jnp.int32)
                    v = plsc.load_gather(rows_v, [row_ids, col])
                    plsc.store_scatter(
                        rows_v, [row_ids, col], v * scale, mask=bad
                    )
                    return c

                lax.fori_loop(0, d, fix_col, 0)


@functools.partial(jax.jit, static_argnums=(3, 4))
def _gather_call(table, idx, mask, n, d):
    per_worker = n // NUM_WORKERS
    n_chunks = per_worker // CHUNK
    mesh = plsc.VectorSubcoreMesh(core_axis_name="c", subcore_axis_name="s")

    nbuf = 5
    assert n_chunks % nbuf == 0

    @functools.partial(
        pl.kernel,
        out_type=jax.ShapeDtypeStruct((n, d), jnp.float32),
        mesh=mesh,
        scratch_types=[
            pltpu.VMEM((per_worker,), jnp.int32),
            pltpu.VMEM((per_worker,), jnp.float32),
            [pltpu.VMEM((CHUNK, d), jnp.float32)] * nbuf,
            [pltpu.SemaphoreType.DMA] * nbuf,
            [pltpu.SemaphoreType.DMA] * nbuf,
        ],
        compiler_params=pltpu.CompilerParams(needs_layout_passes=False),
    )
    def body(table_hbm, idx_hbm, mask_hbm, out_hbm,
             idx_v, mask_v, bufs, gsems, osems):
        wid = lax.axis_index("c") * NUM_SUBCORES + lax.axis_index("s")
        base = wid * per_worker

        pltpu.sync_copy(idx_hbm.at[pl.ds(base, per_worker)], idx_v)
        pltpu.sync_copy(mask_hbm.at[pl.ds(base, per_worker)], mask_v)

        for b in range(nbuf - 1):
            pltpu.async_copy(table_hbm.at[idx_v.at[pl.ds(b * CHUNK, CHUNK)]], bufs[b], gsems[b])

        @pl.loop(0, n_chunks, step=nbuf)
        def _outer(i):
            for b in range(nbuf):
                c = i + b

                b2 = (b + nbuf - 1) % nbuf

                @pl.when(c + nbuf - 1 < n_chunks)
                def _start(c=c, b2=b2):
                    @pl.when(c >= 1)
                    def _wait_wb():
                        pltpu.make_async_copy(
                            bufs[b2],
                            out_hbm.at[pl.ds(base + (c - 1) * CHUNK, CHUNK)],
                            osems[b2],
                        ).wait()

                    pltpu.async_copy(
                        table_hbm.at[idx_v.at[pl.ds((c + nbuf - 1) * CHUNK, CHUNK)]],
                        bufs[b2], gsems[b2],
                    )

                pltpu.make_async_copy(
                    table_hbm.at[idx_v.at[pl.ds(c * CHUNK, CHUNK)]], bufs[b], gsems[b]
                ).wait()

                _fixup(bufs[b], idx_v.at[pl.ds(c * CHUNK, CHUNK)],
                       mask_v.at[pl.ds(c * CHUNK, CHUNK)], d)

                pltpu.async_copy(
                    bufs[b], out_hbm.at[pl.ds(base + c * CHUNK, CHUNK)],
                    osems[b],
                )

        for b in range(nbuf):
            c_last = n_chunks - nbuf + b
            pltpu.make_async_copy(
                bufs[b], out_hbm.at[pl.ds(base + c_last * CHUNK, CHUNK)],
                osems[b],
            ).wait()

    return body(table, idx, mask)


def kernel(input_ids, attention_mask, table):
    b, l = input_ids.shape
    d = table.shape[1]
    n = b * l
    idx = input_ids.reshape(n).astype(jnp.int32)
    mask = attention_mask.reshape(n).astype(jnp.float32)
    out = _gather_call(table, idx, mask, n, d)
    return out.reshape(b, l, d)
